# baseline (speedup 1.0000x reference)
import functools

import jax
import jax.numpy as jnp
from jax import lax
from jax.experimental import pallas as pl
from jax.experimental.pallas import tpu as pltpu
from jax.experimental.pallas import tpu_sc as plsc

_N = 100000
_C = 512
_NC = 2
_NS = 16
_NW = _NC * _NS
_L = 16
_CHUNK = 3136
_TAIL = _N - (_NW - 1) * _CHUNK
_VECS = _CHUNK // _L
_TVECS = _TAIL // _L
_RED = 4
_COLS = _C // _RED

_mesh = plsc.VectorSubcoreMesh(core_axis_name="c", subcore_axis_name="s",
                               num_cores=_NC, num_subcores=_NS)


@functools.partial(
    pl.kernel,
    out_type=jax.ShapeDtypeStruct((_NC, _C), jnp.float32),
    mesh=_mesh,
    scratch_types=[
        pltpu.VMEM((_CHUNK,), jnp.float32),
        pltpu.VMEM((_CHUNK + _L,), jnp.int32),
        pltpu.VMEM((_C,), jnp.float32),
        pltpu.VMEM((_NS, _COLS), jnp.float32),
        pltpu.VMEM((_COLS,), jnp.float32),
        pltpu.VMEM_SHARED((_NS, _C), jnp.float32),
        pltpu.SemaphoreType.DMA,
        pltpu.SemaphoreType.DMA,
    ],
    compiler_params=pltpu.CompilerParams(needs_layout_passes=False),
)
def _seg_sum_sc(inputs_hbm, labels_hbm, out_hbm,
                vals_v, labs_v, acc_v, all_v, out_v, shared, sem_a, sem_b):
    cid = lax.axis_index("c")
    sid = lax.axis_index("s")
    wid = sid * _NC + cid
    base = wid * _CHUNK
    is_tail = wid == _NW - 1

    zeros = jnp.zeros((_L,), jnp.float32)

    def zbody(j, carry):
        acc_v[pl.ds(j * _L, _L)] = zeros
        return carry

    not_top = jnp.arange(_L, dtype=jnp.int32) < _L - 1
    is_top = jnp.logical_not(not_top)

    def body(i):
        lv = labs_v[pl.ds(i, _L)]
        vv = vals_v[pl.ds(i, _L)]
        lv1 = labs_v[pl.ds(i + 1, _L)]
        c = plsc.cumsum(vv)
        boundary = lv != lv1
        plsc.addupdate_scatter(acc_v, [lv], c,
                               mask=jnp.logical_or(boundary, is_top))
        plsc.addupdate_scatter(acc_v, [lv1], -c,
                               mask=jnp.logical_and(boundary, not_top))

    half = _CHUNK // 2
    quart = _CHUNK // 4
    thalf = _TAIL // 2

    @pl.when(jnp.logical_not(is_tail))
    def _():
        a = pltpu.async_copy(inputs_hbm.at[pl.ds(base, quart)],
                             vals_v.at[pl.ds(0, quart)], sem_a)
        b = pltpu.async_copy(labels_hbm.at[pl.ds(base, quart)],
                             labs_v.at[pl.ds(0, quart)], sem_b)
        lax.fori_loop(0, _C // _L, zbody, 0)
        a.wait()
        b.wait()
        rest = _CHUNK - quart
        a2 = pltpu.async_copy(inputs_hbm.at[pl.ds(base + quart, rest)],
                              vals_v.at[pl.ds(quart, rest)], sem_a)
        b2 = pltpu.async_copy(labels_hbm.at[pl.ds(base + quart, rest)],
                              labs_v.at[pl.ds(quart, rest)], sem_b)
        plsc.parallel_loop(0, quart, _L, unroll=4)(body)
        a2.wait()
        b2.wait()
        plsc.parallel_loop(quart, _CHUNK, _L, unroll=4)(body)

    @pl.when(is_tail)
    def _():
        a = pltpu.async_copy(inputs_hbm.at[pl.ds(base, thalf)],
                             vals_v.at[pl.ds(0, thalf)], sem_a)
        b = pltpu.async_copy(labels_hbm.at[pl.ds(base, thalf)],
                             labs_v.at[pl.ds(0, thalf)], sem_b)
        lax.fori_loop(0, _C // _L, zbody, 0)
        a.wait()
        b.wait()
        a2 = pltpu.async_copy(inputs_hbm.at[pl.ds(base + thalf, thalf)],
                              vals_v.at[pl.ds(thalf, thalf)], sem_a)
        b2 = pltpu.async_copy(labels_hbm.at[pl.ds(base + thalf, thalf)],
                              labs_v.at[pl.ds(thalf, thalf)], sem_b)
        plsc.parallel_loop(0, thalf, _L, unroll=4)(body)
        a2.wait()
        b2.wait()
        plsc.parallel_loop(thalf, _TAIL, _L, unroll=4)(body)

    pltpu.sync_copy(acc_v, shared.at[sid])
    plsc.subcore_barrier()

    @pl.when(sid < _RED)
    def _():
        col0 = sid * _COLS
        pltpu.sync_copy(shared.at[:, pl.ds(col0, _COLS)], all_v)
        nacc = _COLS // _L

        def rbody(r, accs):
            return tuple(a + all_v[r, pl.ds(g * _L, _L)]
                         for g, a in enumerate(accs))

        accs = lax.fori_loop(0, _NS, rbody, (zeros,) * nacc)
        for g in range(nacc):
            out_v[pl.ds(g * _L, _L)] = accs[g]
        pltpu.sync_copy(out_v, out_hbm.at[cid, pl.ds(col0, _COLS)])


def kernel(inputs, labels):
    partial = _seg_sum_sc(inputs, labels.astype(jnp.int32))
    return partial[0] + partial[1]


# speedup vs baseline: 1.0048x; 1.0048x over previous
import functools

import jax
import jax.numpy as jnp
from jax import lax
from jax.experimental import pallas as pl
from jax.experimental.pallas import tpu as pltpu
from jax.experimental.pallas import tpu_sc as plsc

_N = 100000
_C = 512
_NC = 2
_NS = 16
_NW = _NC * _NS
_L = 16
_CHUNK = 3136
_TAIL = _N - (_NW - 1) * _CHUNK
_VECS = _CHUNK // _L
_TVECS = _TAIL // _L
_RED = 4
_COLS = _C // _RED

_mesh = plsc.VectorSubcoreMesh(core_axis_name="c", subcore_axis_name="s",
                               num_cores=_NC, num_subcores=_NS)


@functools.partial(
    pl.kernel,
    out_type=jax.ShapeDtypeStruct((_NC, _C), jnp.float32),
    mesh=_mesh,
    scratch_types=[
        pltpu.VMEM((_CHUNK,), jnp.float32),
        pltpu.VMEM((_CHUNK + _L,), jnp.int32),
        pltpu.VMEM((_C,), jnp.float32),
        pltpu.VMEM((_NS, _COLS), jnp.float32),
        pltpu.VMEM((_COLS,), jnp.float32),
        pltpu.VMEM_SHARED((_NS, _C), jnp.float32),
        pltpu.SemaphoreType.DMA,
        pltpu.SemaphoreType.DMA,
    ],
    compiler_params=pltpu.CompilerParams(needs_layout_passes=False),
)
def _seg_sum_sc(inputs_hbm, labels_hbm, out_hbm,
                vals_v, labs_v, acc_v, all_v, out_v, shared, sem_a, sem_b):
    cid = lax.axis_index("c")
    sid = lax.axis_index("s")
    wid = sid * _NC + cid
    base = wid * _CHUNK
    is_tail = wid == _NW - 1

    zeros = jnp.zeros((_L,), jnp.float32)

    def zbody(j, carry):
        acc_v[pl.ds(j * _L, _L)] = zeros
        return carry

    not_top = jnp.arange(_L, dtype=jnp.int32) < _L - 1
    is_top = jnp.logical_not(not_top)

    def body(i):
        lv = labs_v[pl.ds(i, _L)]
        vv = vals_v[pl.ds(i, _L)]
        lv1 = labs_v[pl.ds(i + 1, _L)]
        c = plsc.cumsum(vv)
        boundary = lv != lv1
        plsc.addupdate_scatter(acc_v, [lv], c,
                               mask=jnp.logical_or(boundary, is_top))
        plsc.addupdate_scatter(acc_v, [lv1], -c,
                               mask=jnp.logical_and(boundary, not_top))

    half = _CHUNK // 2
    thalf = _TAIL // 2

    @pl.when(jnp.logical_not(is_tail))
    def _():
        a = pltpu.async_copy(inputs_hbm.at[pl.ds(base, half)],
                             vals_v.at[pl.ds(0, half)], sem_a)
        b = pltpu.async_copy(labels_hbm.at[pl.ds(base, half)],
                             labs_v.at[pl.ds(0, half)], sem_b)
        lax.fori_loop(0, _C // _L, zbody, 0)
        a.wait()
        b.wait()
        a2 = pltpu.async_copy(inputs_hbm.at[pl.ds(base + half, half)],
                              vals_v.at[pl.ds(half, half)], sem_a)
        b2 = pltpu.async_copy(labels_hbm.at[pl.ds(base + half, half)],
                              labs_v.at[pl.ds(half, half)], sem_b)
        plsc.parallel_loop(0, half, _L, unroll=4)(body)
        a2.wait()
        b2.wait()
        plsc.parallel_loop(half, _CHUNK, _L, unroll=4)(body)

    @pl.when(is_tail)
    def _():
        a = pltpu.async_copy(inputs_hbm.at[pl.ds(base, thalf)],
                             vals_v.at[pl.ds(0, thalf)], sem_a)
        b = pltpu.async_copy(labels_hbm.at[pl.ds(base, thalf)],
                             labs_v.at[pl.ds(0, thalf)], sem_b)
        lax.fori_loop(0, _C // _L, zbody, 0)
        a.wait()
        b.wait()
        a2 = pltpu.async_copy(inputs_hbm.at[pl.ds(base + thalf, thalf)],
                              vals_v.at[pl.ds(thalf, thalf)], sem_a)
        b2 = pltpu.async_copy(labels_hbm.at[pl.ds(base + thalf, thalf)],
                              labs_v.at[pl.ds(thalf, thalf)], sem_b)
        plsc.parallel_loop(0, thalf, _L, unroll=4)(body)
        a2.wait()
        b2.wait()
        plsc.parallel_loop(thalf, _TAIL, _L, unroll=4)(body)

    pltpu.sync_copy(acc_v, shared.at[sid])
    plsc.subcore_barrier()

    @pl.when(sid < _RED)
    def _():
        col0 = sid * _COLS
        pltpu.sync_copy(shared.at[:, pl.ds(col0, _COLS)], all_v)
        nacc = _COLS // _L

        def rbody(r, accs):
            return tuple(a + all_v[r, pl.ds(g * _L, _L)]
                         for g, a in enumerate(accs))

        accs = lax.fori_loop(0, _NS, rbody, (zeros,) * nacc)
        for g in range(nacc):
            out_v[pl.ds(g * _L, _L)] = accs[g]
        pltpu.sync_copy(out_v, out_hbm.at[cid, pl.ds(col0, _COLS)])


def kernel(inputs, labels):
    partial = _seg_sum_sc(inputs, labels.astype(jnp.int32))
    return partial[0] + partial[1]
